# baseline (speedup 1.0000x reference)
import jax
import jax.numpy as jnp
from jax import lax
from jax.experimental import pallas as pl
from jax.experimental.pallas import tpu as pltpu


def _make_body(N):
    def _cholec_body(gp_ref, gt_ref, o_ref, gpm_ref, w_ref):
        n = pl.program_id(0)
        P, H, W = gpm_ref.shape
        T = gt_ref.shape[1]

        @pl.when(n > 0)
        def _():
            gp_prev = gpm_ref[...]
            w_prev = w_ref[0, :]
            pas = jnp.sum(gp_prev * w_prev[:, None, None], axis=0)
            cover = jnp.sum(gp_prev, axis=0)
            o_ref[0] = jnp.where(cover > 0.0, pas / cover, 0.0)

        @pl.when(n < N)
        def _():
            gp_m = (gp_ref[0] != 0).astype(jnp.float32)
            gt_m = (gt_ref[0] != 0).astype(jnp.float32)

            acc = jnp.zeros((P * 8, T * 8), jnp.float32)
            for c in range(H // 8):
                a_c = gp_m[:, 8 * c:8 * c + 8, :].reshape(P * 8, W)
                b_c = gt_m[:, 8 * c:8 * c + 8, :].reshape(T * 8, W)
                acc = acc + lax.dot_general(
                    a_c.astype(jnp.bfloat16), b_c.astype(jnp.bfloat16),
                    (((1,), (1,)), ((), ())),
                    preferred_element_type=jnp.float32)

            ph = lax.broadcasted_iota(jnp.int32, (P * 8, T * 8), 0) % 8
            th = lax.broadcasted_iota(jnp.int32, (P * 8, T * 8), 1) % 8
            accm = jnp.where(ph == th, acc, 0.0)
            s2 = accm.reshape(P, 8, T * 8).sum(axis=1)
            fold = (lax.broadcasted_iota(jnp.int32, (T * 8, T), 0) // 8
                    == lax.broadcasted_iota(jnp.int32, (T * 8, T), 1)
                    ).astype(jnp.float32)
            inters = lax.dot_general(
                s2, fold, (((1,), (0,)), ((), ())),
                preferred_element_type=jnp.float32)

            area = jnp.sum(gt_m, axis=(1, 2))
            safe = jnp.where(area > 0.0, area, 1.0)
            iogs = jnp.where(area[None, :] > 0.0, inters / safe[None, :], 0.0)
            iog_max = jnp.max(iogs, axis=1)

            gpm_ref[...] = gp_m
            w_ref[...] = lax.broadcast_in_dim(iog_max, (8, P), (1,))

    return _cholec_body


def kernel(groups_pred, groups_true):
    N, P, H, W = groups_pred.shape
    T = groups_true.shape[1]

    return pl.pallas_call(
        _make_body(N),
        grid=(N + 1,),
        in_specs=[
            pl.BlockSpec((1, P, H, W),
                         lambda n: (jnp.minimum(n, N - 1), 0, 0, 0)),
            pl.BlockSpec((1, T, H, W),
                         lambda n: (jnp.minimum(n, N - 1), 0, 0, 0)),
        ],
        out_specs=pl.BlockSpec((1, H, W),
                               lambda n: (jnp.maximum(n - 1, 0), 0, 0)),
        out_shape=jax.ShapeDtypeStruct((N, H, W), jnp.float32),
        scratch_shapes=[
            pltpu.VMEM((P, H, W), jnp.float32),
            pltpu.VMEM((8, P), jnp.float32),
        ],
        compiler_params=pltpu.CompilerParams(
            dimension_semantics=("arbitrary",),
            vmem_limit_bytes=56 * 1024 * 1024,
        ),
        name="cholec_metric",
    )(groups_pred, groups_true)
